# Initial kernel scaffold; baseline (speedup 1.0000x reference)
import jax
import jax.numpy as jnp
from jax.experimental import pallas as pl


def kernel(x, pos, norm, batch, params):
    raise NotImplementedError("write your pallas kernel here")


# speedup vs baseline: 1.1387x; 1.1387x over previous
import functools
import math

import jax
import jax.numpy as jnp
from jax.experimental import pallas as pl
from jax.experimental.pallas import tpu as pltpu

_NHID = 32
_DEPTH = 3
_RATIO = 0.5
_RADIUS = 2.0
_K_NBR = 64
_KNN_K = 3
_EPS = 1e-5


def _linrelu_body(x_ref, w_ref, b_ref, o_ref):
    o_ref[...] = jnp.maximum(
        jnp.dot(x_ref[...], w_ref[...], preferred_element_type=jnp.float32)
        + b_ref[...],
        0.0,
    )


def _linrelu(x, w, b):
    n, _ = x.shape
    cout = w.shape[1]
    return pl.pallas_call(
        _linrelu_body,
        out_shape=jax.ShapeDtypeStruct((n, cout), jnp.float32),
    )(x, w, b.reshape(1, -1))


def _mlp2_body(h_ref, w1_ref, b1_ref, g1_ref, be1_ref, w2_ref, b2_ref,
               g2_ref, be2_ref, o_ref):
    h = h_ref[...]
    h = jnp.maximum(
        jnp.dot(h, w1_ref[...], preferred_element_type=jnp.float32)
        + b1_ref[...], 0.0)
    mean = jnp.mean(h, axis=0, keepdims=True)
    var = jnp.mean((h - mean) ** 2, axis=0, keepdims=True)
    h = (h - mean) / jnp.sqrt(var + _EPS) * g1_ref[...] + be1_ref[...]
    h = jnp.maximum(
        jnp.dot(h, w2_ref[...], preferred_element_type=jnp.float32)
        + b2_ref[...], 0.0)
    mean = jnp.mean(h, axis=0, keepdims=True)
    var = jnp.mean((h - mean) ** 2, axis=0, keepdims=True)
    o_ref[...] = (h - mean) / jnp.sqrt(var + _EPS) * g2_ref[...] + be2_ref[...]


def _mlp2_pallas(h, p):
    n = h.shape[0]
    cout = p["w2"].shape[1]
    return pl.pallas_call(
        _mlp2_body,
        out_shape=jax.ShapeDtypeStruct((n, cout), jnp.float32),
    )(h, p["w1"], p["b1"].reshape(1, -1), p["g1"].reshape(1, -1),
      p["be1"].reshape(1, -1), p["w2"], p["b2"].reshape(1, -1),
      p["g2"].reshape(1, -1), p["be2"].reshape(1, -1))


def _pairwise_d2(a, b):
    d2 = (jnp.sum(a * a, 1)[:, None] + jnp.sum(b * b, 1)[None, :]
          - 2.0 * (a @ b.T))
    return jnp.maximum(d2, 0.0)


def _fps(pos, npoints):
    d0 = jnp.sum((pos - pos[0]) ** 2, axis=1)
    sel0 = jnp.zeros((npoints,), jnp.int32)

    def body(i, st):
        sel, dmin = st
        nxt = jnp.argmax(dmin).astype(jnp.int32)
        sel = sel.at[i].set(nxt)
        d = jnp.sum((pos - pos[nxt]) ** 2, axis=1)
        return (sel, jnp.minimum(dmin, d))

    sel, _ = jax.lax.fori_loop(1, npoints, body, (sel0, d0))
    return sel


def _radius_knn(pos_src, pos_dst, r, k):
    d2 = _pairwise_d2(pos_dst, pos_src)
    neg = jnp.where(d2 <= r * r, -d2, -jnp.inf)
    vals, idx = jax.lax.top_k(neg, k)
    valid = jnp.isfinite(vals)
    idx = jnp.where(valid, idx, 0)
    return idx, valid


def _bn_masked(h, mask, g, be):
    w = mask.astype(h.dtype)
    s = jnp.maximum(jnp.sum(w), 1.0)
    mean = jnp.sum(h * w[:, None], 0) / s
    var = jnp.sum(((h - mean) ** 2) * w[:, None], 0) / s
    return (h - mean) / jnp.sqrt(var + _EPS) * g + be


def _mlp2_masked(h, mask, p):
    h = jnp.maximum(h @ p["w1"] + p["b1"], 0.0)
    h = _bn_masked(h, mask, p["g1"], p["be1"])
    h = jnp.maximum(h @ p["w2"] + p["b2"], 0.0)
    h = _bn_masked(h, mask, p["g2"], p["be2"])
    return h


def _sa_stage(x, pos, p, npoints):
    idx = _fps(pos, npoints)
    pos_c = pos[idx]
    nbr, valid = _radius_knn(pos, pos_c, _RADIUS, _K_NBR)
    x_j = x[nbr]
    rel = pos[nbr] - pos_c[:, None, :]
    h = jnp.concatenate([x_j, rel], axis=-1)
    m, k, c = h.shape
    hf = _mlp2_masked(h.reshape(m * k, c), valid.reshape(m * k), p)
    h = hf.reshape(m, k, -1)
    h = jnp.where(valid[:, :, None], h, -jnp.inf)
    agg = jnp.max(h, axis=1)
    agg = jnp.where(jnp.any(valid, axis=1)[:, None], agg, 0.0)
    return agg, pos_c


def _fp_stage(x, pos, x_skip, pos_skip, p):
    d2 = _pairwise_d2(pos_skip, pos)
    vals, idx = jax.lax.top_k(-d2, _KNN_K)
    w = 1.0 / jnp.maximum(-vals, 1e-16)
    xi = jnp.sum(x[idx] * w[:, :, None], axis=1) / jnp.sum(w, axis=1)[:, None]
    h = jnp.concatenate([xi, x_skip], axis=1)
    return _mlp2_pallas(h, p)


def kernel(x, pos, norm, batch, params):
    p = params
    h = _linrelu(x, p["lin_in"]["w"], p["lin_in"]["b"])
    sa = [(h, pos)]
    for i in range(_DEPTH):
        hi, posi = sa[i]
        npts = int(math.ceil(_RATIO * posi.shape[0]))
        ho, po = _sa_stage(hi, posi, p["sa"][i], npts)
        sa.append((ho, po))
    fx, fpos = sa[_DEPTH]
    for i in range(_DEPTH):
        sx, spos = sa[_DEPTH - 1 - i]
        fx = _fp_stage(fx, fpos, sx, spos, p["fp"][_DEPTH - 1 - i])
        fpos = spos
    return _mlp2_pallas(fx, p["out"])


# speedup vs baseline: 2.9993x; 2.6340x over previous
import functools
import math

import jax
import jax.numpy as jnp
from jax.experimental import pallas as pl
from jax.experimental.pallas import tpu as pltpu

_NHID = 32
_DEPTH = 3
_RATIO = 0.5
_RADIUS = 2.0
_K_NBR = 64
_KNN_K = 3
_EPS = 1e-5


def _linrelu_body(x_ref, w_ref, b_ref, o_ref):
    o_ref[...] = jnp.maximum(
        jnp.dot(x_ref[...], w_ref[...], preferred_element_type=jnp.float32)
        + b_ref[...],
        0.0,
    )


def _linrelu(x, w, b):
    n, _ = x.shape
    cout = w.shape[1]
    return pl.pallas_call(
        _linrelu_body,
        out_shape=jax.ShapeDtypeStruct((n, cout), jnp.float32),
    )(x, w, b.reshape(1, -1))


def _mlp2_body(h_ref, w1_ref, b1_ref, g1_ref, be1_ref, w2_ref, b2_ref,
               g2_ref, be2_ref, o_ref):
    h = h_ref[...]
    h = jnp.maximum(
        jnp.dot(h, w1_ref[...], preferred_element_type=jnp.float32)
        + b1_ref[...], 0.0)
    mean = jnp.mean(h, axis=0, keepdims=True)
    var = jnp.mean((h - mean) ** 2, axis=0, keepdims=True)
    h = (h - mean) / jnp.sqrt(var + _EPS) * g1_ref[...] + be1_ref[...]
    h = jnp.maximum(
        jnp.dot(h, w2_ref[...], preferred_element_type=jnp.float32)
        + b2_ref[...], 0.0)
    mean = jnp.mean(h, axis=0, keepdims=True)
    var = jnp.mean((h - mean) ** 2, axis=0, keepdims=True)
    o_ref[...] = (h - mean) / jnp.sqrt(var + _EPS) * g2_ref[...] + be2_ref[...]


def _mlp2_pallas(h, p):
    n = h.shape[0]
    cout = p["w2"].shape[1]
    return pl.pallas_call(
        _mlp2_body,
        out_shape=jax.ShapeDtypeStruct((n, cout), jnp.float32),
    )(h, p["w1"], p["b1"].reshape(1, -1), p["g1"].reshape(1, -1),
      p["be1"].reshape(1, -1), p["w2"], p["b2"].reshape(1, -1),
      p["g2"].reshape(1, -1), p["be2"].reshape(1, -1))


def _fps_body(pos_ref, sel_ref, px_ref, py_ref, pz_ref, *, n, m):
    r = pos_ref.shape[1]
    px = pos_ref[0]
    py = pos_ref[1]
    pz = pos_ref[2]
    row = jax.lax.broadcasted_iota(jnp.int32, (r, 128), 0)
    lane = jax.lax.broadcasted_iota(jnp.int32, (r, 128), 1)
    gidx = row * 128 + lane
    valid = gidx < n
    m0 = gidx == 0
    qx = jnp.sum(jnp.where(m0, px, 0.0))
    qy = jnp.sum(jnp.where(m0, py, 0.0))
    qz = jnp.sum(jnp.where(m0, pz, 0.0))
    sel_ref[0] = 0
    px_ref[0] = qx
    py_ref[0] = qy
    pz_ref[0] = qz
    d0 = (px - qx) ** 2 + (py - qy) ** 2 + (pz - qz) ** 2
    dmin0 = jnp.where(valid, d0, -1.0)

    def body(i, dmin):
        mx = jnp.max(dmin)
        nxt = jnp.min(jnp.where(dmin == mx, gidx, jnp.int32(2 ** 30)))
        sel_ref[i] = nxt
        nm = gidx == nxt
        nqx = jnp.sum(jnp.where(nm, px, 0.0))
        nqy = jnp.sum(jnp.where(nm, py, 0.0))
        nqz = jnp.sum(jnp.where(nm, pz, 0.0))
        px_ref[i] = nqx
        py_ref[i] = nqy
        pz_ref[i] = nqz
        d = (px - nqx) ** 2 + (py - nqy) ** 2 + (pz - nqz) ** 2
        return jnp.where(valid, jnp.minimum(dmin, d), -1.0)

    jax.lax.fori_loop(1, m, body, dmin0)


def _fps_pallas(pos, npoints):
    n = pos.shape[0]
    npad = ((n + 127) // 128) * 128
    r = npad // 128
    post = jnp.pad(pos, ((0, npad - n), (0, 0))).T.reshape(3, r, 128)
    f32 = jnp.float32
    sel, cx, cy, cz = pl.pallas_call(
        functools.partial(_fps_body, n=n, m=npoints),
        out_shape=[
            jax.ShapeDtypeStruct((npoints,), jnp.int32),
            jax.ShapeDtypeStruct((npoints,), f32),
            jax.ShapeDtypeStruct((npoints,), f32),
            jax.ShapeDtypeStruct((npoints,), f32),
        ],
        out_specs=[pl.BlockSpec(memory_space=pltpu.SMEM)] * 4,
    )(post)
    return sel, jnp.stack([cx, cy, cz], axis=1)


def _pairwise_d2(a, b):
    d2 = (jnp.sum(a * a, 1)[:, None] + jnp.sum(b * b, 1)[None, :]
          - 2.0 * (a @ b.T))
    return jnp.maximum(d2, 0.0)


def _fps(pos, npoints):
    d0 = jnp.sum((pos - pos[0]) ** 2, axis=1)
    sel0 = jnp.zeros((npoints,), jnp.int32)

    def body(i, st):
        sel, dmin = st
        nxt = jnp.argmax(dmin).astype(jnp.int32)
        sel = sel.at[i].set(nxt)
        d = jnp.sum((pos - pos[nxt]) ** 2, axis=1)
        return (sel, jnp.minimum(dmin, d))

    sel, _ = jax.lax.fori_loop(1, npoints, body, (sel0, d0))
    return sel


def _radius_knn(pos_src, pos_dst, r, k):
    d2 = _pairwise_d2(pos_dst, pos_src)
    neg = jnp.where(d2 <= r * r, -d2, -jnp.inf)
    vals, idx = jax.lax.top_k(neg, k)
    valid = jnp.isfinite(vals)
    idx = jnp.where(valid, idx, 0)
    return idx, valid


def _bn_masked(h, mask, g, be):
    w = mask.astype(h.dtype)
    s = jnp.maximum(jnp.sum(w), 1.0)
    mean = jnp.sum(h * w[:, None], 0) / s
    var = jnp.sum(((h - mean) ** 2) * w[:, None], 0) / s
    return (h - mean) / jnp.sqrt(var + _EPS) * g + be


def _mlp2_masked(h, mask, p):
    h = jnp.maximum(h @ p["w1"] + p["b1"], 0.0)
    h = _bn_masked(h, mask, p["g1"], p["be1"])
    h = jnp.maximum(h @ p["w2"] + p["b2"], 0.0)
    h = _bn_masked(h, mask, p["g2"], p["be2"])
    return h


def _sa_stage(x, pos, p, npoints):
    idx, pos_c = _fps_pallas(pos, npoints)
    nbr, valid = _radius_knn(pos, pos_c, _RADIUS, _K_NBR)
    x_j = x[nbr]
    rel = pos[nbr] - pos_c[:, None, :]
    h = jnp.concatenate([x_j, rel], axis=-1)
    m, k, c = h.shape
    hf = _mlp2_masked(h.reshape(m * k, c), valid.reshape(m * k), p)
    h = hf.reshape(m, k, -1)
    h = jnp.where(valid[:, :, None], h, -jnp.inf)
    agg = jnp.max(h, axis=1)
    agg = jnp.where(jnp.any(valid, axis=1)[:, None], agg, 0.0)
    return agg, pos_c


def _fp_stage(x, pos, x_skip, pos_skip, p):
    d2 = _pairwise_d2(pos_skip, pos)
    vals, idx = jax.lax.top_k(-d2, _KNN_K)
    w = 1.0 / jnp.maximum(-vals, 1e-16)
    xi = jnp.sum(x[idx] * w[:, :, None], axis=1) / jnp.sum(w, axis=1)[:, None]
    h = jnp.concatenate([xi, x_skip], axis=1)
    return _mlp2_pallas(h, p)


def kernel(x, pos, norm, batch, params):
    p = params
    h = _linrelu(x, p["lin_in"]["w"], p["lin_in"]["b"])
    sa = [(h, pos)]
    for i in range(_DEPTH):
        hi, posi = sa[i]
        npts = int(math.ceil(_RATIO * posi.shape[0]))
        ho, po = _sa_stage(hi, posi, p["sa"][i], npts)
        sa.append((ho, po))
    fx, fpos = sa[_DEPTH]
    for i in range(_DEPTH):
        sx, spos = sa[_DEPTH - 1 - i]
        fx = _fp_stage(fx, fpos, sx, spos, p["fp"][_DEPTH - 1 - i])
        fpos = spos
    return _mlp2_pallas(fx, p["out"])
